# baseline (speedup 1.0000x reference)
import functools

import jax
import jax.numpy as jnp
from jax import lax
from jax.experimental import pallas as pl
from jax.experimental.pallas import tpu as pltpu
from jax.experimental.pallas import tpu_sc as plsc

_NUM_CORES = 2
_NUM_SUBCORES = 16
_NUM_WORKERS = _NUM_CORES * _NUM_SUBCORES


def _sc_gather(table, idx):
    _, d = table.shape
    b = idx.shape[0]
    b_per_w = b // _NUM_WORKERS
    mesh = plsc.VectorSubcoreMesh(core_axis_name="c", subcore_axis_name="s")

    @functools.partial(
        pl.kernel,
        out_type=jax.ShapeDtypeStruct((b, d), jnp.float32),
        mesh=mesh,
        scratch_types=[
            pltpu.VMEM((b_per_w,), jnp.int32),
            pltpu.VMEM((b_per_w, d), jnp.float32),
            pltpu.SemaphoreType.DMA,
        ],
        compiler_params=pltpu.CompilerParams(use_tc_tiling_on_sc=False),
    )
    def gather_kernel(table_hbm, idx_hbm, out_hbm, idx_v, rows_v, sem):
        wid = lax.axis_index("s") * _NUM_CORES + lax.axis_index("c")
        base = wid * b_per_w
        pltpu.sync_copy(idx_hbm.at[pl.ds(base, b_per_w)], idx_v)
        pltpu.async_copy(table_hbm.at[idx_v], rows_v, sem).wait()
        pltpu.sync_copy(rows_v, out_hbm.at[pl.ds(base, b_per_w)])

    return gather_kernel(table, idx)


def _tc_expand(bw, aspects, block_b=1024):
    b, a = bw.shape
    _, d = aspects.shape

    def body(w_ref, a_ref, o_ref):
        o_ref[...] = w_ref[...][:, :, None] * a_ref[...][None, :, :]

    return pl.pallas_call(
        body,
        grid=(b // block_b,),
        in_specs=[
            pl.BlockSpec((block_b, a), lambda i: (i, 0)),
            pl.BlockSpec((a, d), lambda i: (0, 0)),
        ],
        out_specs=pl.BlockSpec((block_b, a, d), lambda i: (i, 0, 0)),
        out_shape=jax.ShapeDtypeStruct((b, a, d), jnp.float32),
        compiler_params=pltpu.CompilerParams(
            dimension_semantics=("arbitrary",),
        ),
    )(bw, aspects)


def kernel(data, brand_table, aspects):
    brand_weights = _sc_gather(brand_table, data.astype(jnp.int32))
    return _tc_expand(brand_weights, aspects)


# speedup vs baseline: 2.6544x; 2.6544x over previous
import functools

import jax
import jax.numpy as jnp
from jax import lax
from jax.experimental import pallas as pl
from jax.experimental.pallas import tpu as pltpu
from jax.experimental.pallas import tpu_sc as plsc

_NUM_CORES = 2
_NUM_SUBCORES = 16
_NUM_WORKERS = _NUM_CORES * _NUM_SUBCORES


def _sc_gather(table, idx):
    _, d = table.shape
    b = idx.shape[0]
    b_per_w = b // _NUM_WORKERS
    mesh = plsc.VectorSubcoreMesh(core_axis_name="c", subcore_axis_name="s")

    @functools.partial(
        pl.kernel,
        out_type=jax.ShapeDtypeStruct((b, d), jnp.float32),
        mesh=mesh,
        scratch_types=[
            pltpu.VMEM((b_per_w,), jnp.int32),
            pltpu.VMEM((b_per_w, d), jnp.float32),
            pltpu.SemaphoreType.DMA,
        ],
        compiler_params=pltpu.CompilerParams(use_tc_tiling_on_sc=False),
    )
    def gather_kernel(table_hbm, idx_hbm, out_hbm, idx_v, rows_v, sem):
        wid = lax.axis_index("s") * _NUM_CORES + lax.axis_index("c")
        base = wid * b_per_w
        pltpu.sync_copy(idx_hbm.at[pl.ds(base, b_per_w)], idx_v)
        pltpu.async_copy(table_hbm.at[idx_v], rows_v, sem).wait()
        pltpu.sync_copy(rows_v, out_hbm.at[pl.ds(base, b_per_w)])

    return gather_kernel(table, idx)


def _tc_expand_t(a_mat, w_t, block_b=2048):
    r, a = a_mat.shape
    _, b = w_t.shape

    def body(a_ref, w_ref, o_ref):
        o_ref[...] = jnp.dot(
            a_ref[...], w_ref[...], preferred_element_type=jnp.float32
        )

    return pl.pallas_call(
        body,
        grid=(b // block_b,),
        in_specs=[
            pl.BlockSpec((r, a), lambda i: (0, 0)),
            pl.BlockSpec((a, block_b), lambda i: (0, i)),
        ],
        out_specs=pl.BlockSpec((r, block_b), lambda i: (0, i)),
        out_shape=jax.ShapeDtypeStruct((r, b), jnp.float32),
        compiler_params=pltpu.CompilerParams(
            dimension_semantics=("arbitrary",),
        ),
    )(a_mat, w_t)


def kernel(data, brand_table, aspects):
    a, d = aspects.shape
    brand_weights = _sc_gather(brand_table, data.astype(jnp.int32))
    w_t = brand_weights.T
    rows = jnp.arange(a * d, dtype=jnp.int32)
    cols = jnp.arange(a, dtype=jnp.int32)
    a_mat = jnp.where(
        cols[None, :] == (rows // d)[:, None],
        aspects.reshape(-1)[:, None],
        0.0,
    )
    out2 = _tc_expand_t(a_mat, w_t)
    b = data.shape[0]
    return out2.reshape(a, d, b).transpose(2, 0, 1)


# speedup vs baseline: 3.4027x; 1.2819x over previous
import functools

import jax
import jax.numpy as jnp
from jax import lax
from jax.experimental import pallas as pl
from jax.experimental.pallas import tpu as pltpu
from jax.experimental.pallas import tpu_sc as plsc

_NUM_CORES = 2
_NUM_SUBCORES = 16
_NUM_WORKERS = _NUM_CORES * _NUM_SUBCORES


def _sc_gather_t(table_t, idx, chunk=4096):
    a, v = table_t.shape
    b = idx.shape[0]
    n_chunks = b // chunk
    g = chunk // 16
    mesh = plsc.VectorSubcoreMesh(core_axis_name="c", subcore_axis_name="s")

    @functools.partial(
        pl.kernel,
        out_type=jax.ShapeDtypeStruct((a, b), jnp.float32),
        mesh=mesh,
        scratch_types=[
            pltpu.VMEM((v,), jnp.float32),
            pltpu.VMEM((chunk,), jnp.int32),
            pltpu.VMEM((chunk,), jnp.float32),
        ],
        compiler_params=pltpu.CompilerParams(
            use_tc_tiling_on_sc=False, needs_layout_passes=False
        ),
    )
    def gather_kernel(table_hbm, idx_hbm, out_hbm, row_v, idx_v, out_v):
        wid = lax.axis_index("s") * _NUM_CORES + lax.axis_index("c")
        pltpu.sync_copy(table_hbm.at[wid], row_v)
        for c in range(n_chunks):
            pltpu.sync_copy(idx_hbm.at[pl.ds(c * chunk, chunk)], idx_v)

            def body(j, carry):
                vals = plsc.load_gather(row_v, [idx_v[pl.ds(j * 16, 16)]])
                out_v[pl.ds(j * 16, 16)] = vals
                return carry

            lax.fori_loop(0, g, body, 0)
            pltpu.sync_copy(out_v, out_hbm.at[wid, pl.ds(c * chunk, chunk)])

    return gather_kernel(table_t, idx)


def _tc_expand_t(a_mat, w_t, block_b=2048):
    r, a = a_mat.shape
    _, b = w_t.shape

    def body(a_ref, w_ref, o_ref):
        o_ref[...] = jnp.dot(
            a_ref[...], w_ref[...], preferred_element_type=jnp.float32
        )

    return pl.pallas_call(
        body,
        grid=(b // block_b,),
        in_specs=[
            pl.BlockSpec((r, a), lambda i: (0, 0)),
            pl.BlockSpec((a, block_b), lambda i: (0, i)),
        ],
        out_specs=pl.BlockSpec((r, block_b), lambda i: (0, i)),
        out_shape=jax.ShapeDtypeStruct((r, b), jnp.float32),
        compiler_params=pltpu.CompilerParams(
            dimension_semantics=("arbitrary",),
        ),
    )(a_mat, w_t)


def kernel(data, brand_table, aspects):
    a, d = aspects.shape
    w_t = _sc_gather_t(brand_table.T, data.astype(jnp.int32))
    rows = jnp.arange(a * d, dtype=jnp.int32)
    cols = jnp.arange(a, dtype=jnp.int32)
    a_mat = jnp.where(
        cols[None, :] == (rows // d)[:, None],
        aspects.reshape(-1)[:, None],
        0.0,
    )
    out2 = _tc_expand_t(a_mat, w_t)
    b = data.shape[0]
    return out2.reshape(a, d, b).transpose(2, 0, 1)


# speedup vs baseline: 4.1211x; 1.2111x over previous
import functools

import jax
import jax.numpy as jnp
from jax import lax
from jax.experimental import pallas as pl
from jax.experimental.pallas import tpu as pltpu
from jax.experimental.pallas import tpu_sc as plsc

_NUM_CORES = 2
_NUM_SUBCORES = 16
_NUM_WORKERS = _NUM_CORES * _NUM_SUBCORES


def _sc_gather_t(table_t, idx, chunk=4096):
    a, v = table_t.shape
    b = idx.shape[0]
    n_chunks = b // chunk
    g = chunk // 16
    mesh = plsc.VectorSubcoreMesh(core_axis_name="c", subcore_axis_name="s")

    @functools.partial(
        pl.kernel,
        out_type=jax.ShapeDtypeStruct((a, b), jnp.float32),
        mesh=mesh,
        scratch_types=[
            pltpu.VMEM((v,), jnp.float32),
            pltpu.VMEM((chunk,), jnp.int32),
            pltpu.VMEM((chunk,), jnp.float32),
        ],
        compiler_params=pltpu.CompilerParams(
            use_tc_tiling_on_sc=True, needs_layout_passes=False
        ),
    )
    def gather_kernel(table_hbm, idx_hbm, out_hbm, row_v, idx_v, out_v):
        wid = lax.axis_index("s") * _NUM_CORES + lax.axis_index("c")
        pltpu.sync_copy(table_hbm.at[wid], row_v)
        for c in range(n_chunks):
            pltpu.sync_copy(idx_hbm.at[pl.ds(c * chunk, chunk)], idx_v)

            def body(j, carry):
                vals = plsc.load_gather(row_v, [idx_v[pl.ds(j * 16, 16)]])
                out_v[pl.ds(j * 16, 16)] = vals
                return carry

            lax.fori_loop(0, g, body, 0)
            pltpu.sync_copy(out_v, out_hbm.at[wid, pl.ds(c * chunk, chunk)])

    return gather_kernel(table_t, idx)


def _tc_expand_t(a_mat, w_t, block_b=2048):
    r, a = a_mat.shape
    _, b = w_t.shape

    def body(a_ref, w_ref, o_ref):
        o_ref[...] = jnp.dot(
            a_ref[...], w_ref[...], preferred_element_type=jnp.float32
        )

    return pl.pallas_call(
        body,
        grid=(b // block_b,),
        in_specs=[
            pl.BlockSpec((r, a), lambda i: (0, 0)),
            pl.BlockSpec((a, block_b), lambda i: (0, i)),
        ],
        out_specs=pl.BlockSpec((r, block_b), lambda i: (0, i)),
        out_shape=jax.ShapeDtypeStruct((r, b), jnp.float32),
        compiler_params=pltpu.CompilerParams(
            dimension_semantics=("arbitrary",),
        ),
    )(a_mat, w_t)


def kernel(data, brand_table, aspects):
    a, d = aspects.shape
    w_t = _sc_gather_t(brand_table.T, data.astype(jnp.int32))
    rows = jnp.arange(a * d, dtype=jnp.int32)
    cols = jnp.arange(a, dtype=jnp.int32)
    a_mat = jnp.where(
        cols[None, :] == (rows // d)[:, None],
        aspects.reshape(-1)[:, None],
        0.0,
    )
    out2 = _tc_expand_t(a_mat, w_t)
    b = data.shape[0]
    return out2.reshape(a, d, b).transpose(2, 0, 1)


# speedup vs baseline: 4.3878x; 1.0647x over previous
import functools

import jax
import jax.numpy as jnp
from jax import lax
from jax.experimental import pallas as pl
from jax.experimental.pallas import tpu as pltpu
from jax.experimental.pallas import tpu_sc as plsc

_NUM_CORES = 2
_NUM_SUBCORES = 16
_NUM_WORKERS = _NUM_CORES * _NUM_SUBCORES


def _sc_gather_t(table_t, idx, chunk=4096):
    a, v = table_t.shape
    b = idx.shape[0]
    n_chunks = b // chunk
    g = chunk // 16
    mesh = plsc.VectorSubcoreMesh(core_axis_name="c", subcore_axis_name="s")

    @functools.partial(
        pl.kernel,
        out_type=jax.ShapeDtypeStruct((a, b), jnp.float32),
        mesh=mesh,
        scratch_types=[
            pltpu.VMEM((v,), jnp.float32),
            pltpu.VMEM((chunk,), jnp.int32),
            pltpu.VMEM((chunk,), jnp.int32),
            pltpu.VMEM((chunk,), jnp.float32),
            pltpu.VMEM((chunk,), jnp.float32),
            pltpu.SemaphoreType.DMA,
            pltpu.SemaphoreType.DMA,
            pltpu.SemaphoreType.DMA,
            pltpu.SemaphoreType.DMA,
            pltpu.SemaphoreType.DMA,
        ],
        compiler_params=pltpu.CompilerParams(
            use_tc_tiling_on_sc=True, needs_layout_passes=False
        ),
    )
    def gather_kernel(table_hbm, idx_hbm, out_hbm, row_v, idx_v0, idx_v1,
                      out_v0, out_v1, sem_row, sem_i0, sem_i1, sem_o0,
                      sem_o1):
        wid = lax.axis_index("s") * _NUM_CORES + lax.axis_index("c")
        idx_bufs = (idx_v0, idx_v1)
        out_bufs = (out_v0, out_v1)
        isems = (sem_i0, sem_i1)
        osems = (sem_o0, sem_o1)

        row_cp = pltpu.async_copy(table_hbm.at[wid], row_v, sem_row)
        idx_cps = [
            pltpu.async_copy(
                idx_hbm.at[pl.ds(c * chunk, chunk)], idx_bufs[c % 2],
                isems[c % 2],
            )
            for c in range(min(2, n_chunks))
        ]
        row_cp.wait()

        out_cps = [None, None]
        for c in range(n_chunks):
            buf = c % 2
            idx_cps[c].wait()
            if out_cps[buf] is not None:
                out_cps[buf].wait()
            idx_b = idx_bufs[buf]
            out_b = out_bufs[buf]

            def body(j, carry, idx_b=idx_b, out_b=out_b):
                base = j * 64
                for k in range(4):
                    vals = plsc.load_gather(
                        row_v, [idx_b[pl.ds(base + k * 16, 16)]]
                    )
                    out_b[pl.ds(base + k * 16, 16)] = vals
                return carry

            lax.fori_loop(0, chunk // 64, body, 0)
            if c + 2 < n_chunks:
                idx_cps.append(
                    pltpu.async_copy(
                        idx_hbm.at[pl.ds((c + 2) * chunk, chunk)],
                        idx_bufs[buf], isems[buf],
                    )
                )
            out_cps[buf] = pltpu.async_copy(
                out_b, out_hbm.at[wid, pl.ds(c * chunk, chunk)], osems[buf]
            )
        for cp in out_cps:
            if cp is not None:
                cp.wait()

    return gather_kernel(table_t, idx)


def _tc_expand_t(a_mat, w_t, block_b=2048):
    r, a = a_mat.shape
    _, b = w_t.shape

    def body(a_ref, w_ref, o_ref):
        o_ref[...] = jnp.dot(
            a_ref[...], w_ref[...], preferred_element_type=jnp.float32
        )

    return pl.pallas_call(
        body,
        grid=(b // block_b,),
        in_specs=[
            pl.BlockSpec((r, a), lambda i: (0, 0)),
            pl.BlockSpec((a, block_b), lambda i: (0, i)),
        ],
        out_specs=pl.BlockSpec((r, block_b), lambda i: (0, i)),
        out_shape=jax.ShapeDtypeStruct((r, b), jnp.float32),
        compiler_params=pltpu.CompilerParams(
            dimension_semantics=("arbitrary",),
        ),
    )(a_mat, w_t)


def kernel(data, brand_table, aspects):
    a, d = aspects.shape
    w_t = _sc_gather_t(brand_table.T, data.astype(jnp.int32))
    rows = jnp.arange(a * d, dtype=jnp.int32)
    cols = jnp.arange(a, dtype=jnp.int32)
    a_mat = jnp.where(
        cols[None, :] == (rows // d)[:, None],
        aspects.reshape(-1)[:, None],
        0.0,
    )
    out2 = _tc_expand_t(a_mat, w_t)
    b = data.shape[0]
    return out2.reshape(a, d, b).transpose(2, 0, 1)
